# baseline (speedup 1.0000x reference)
import functools

import jax
import jax.numpy as jnp
from jax import lax
from jax.experimental import pallas as pl
from jax.experimental.pallas import tpu as pltpu
from jax.experimental.pallas import tpu_sc as plsc

NC = 2
NS = 16
L = 16
NW = NC * NS
B = 16384
F = 32
BPW = B // NW
NCHUNK = 4
CHUNK = BPW // NCHUNK
NGRP = BPW // L


def _rot_idx(lane, k):
  return (lane + k) & (L - 1)


_GATHER_DNUMS = lax.GatherDimensionNumbers(
    offset_dims=(), collapsed_slice_dims=(0,), start_index_map=(0,))


def _take(x, idx):
  return lax.gather(x, idx[:, None], _GATHER_DNUMS, slice_sizes=(1,),
                    mode=lax.GatherScatterMode.PROMISE_IN_BOUNDS)


def _sc_body(user_hbm, item_hbm, uf_hbm, if_hbm, out_hbm,
             uidx_v, iidx_v, urows_v, irows_v, out_v, sem):
  wid = lax.axis_index("s") * NC + lax.axis_index("c")
  base = wid * BPW

  for j in range(NCHUNK):
    pltpu.sync_copy(user_hbm.at[pl.ds(base + j * CHUNK, CHUNK)], uidx_v.at[j])
    pltpu.sync_copy(item_hbm.at[pl.ds(base + j * CHUNK, CHUNK)], iidx_v.at[j])

  copies = []
  for j in range(NCHUNK):
    copies.append(pltpu.async_copy(
        uf_hbm.at[uidx_v.at[j]], urows_v.at[pl.ds(j * CHUNK, CHUNK)], sem))
    copies.append(pltpu.async_copy(
        if_hbm.at[iidx_v.at[j]], irows_v.at[pl.ds(j * CHUNK, CHUNK)], sem))
  for c in copies:
    c.wait()

  lane = lax.iota(jnp.int32, L)
  rot = {k: _rot_idx(lane, k) for k in (1, 2, 4, 8)}
  rotr = {k: _rot_idx(lane, L - k) for k in (1, 2, 4)}
  keep = {k: (lane & k) == 0 for k in (1, 2, 4, 8)}
  perm = (((lane & 1) << 3) | ((lane & 2) << 1) |
          ((lane & 4) >> 1) | ((lane & 8) >> 3))

  def group(g, _):
    r0 = g * L
    t = []
    for i in range(L):
      r = r0 + i
      u0 = urows_v[r, pl.ds(0, L)]
      u1 = urows_v[r, pl.ds(L, L)]
      v0 = irows_v[r, pl.ds(0, L)]
      v1 = irows_v[r, pl.ds(L, L)]
      t.append(u0 * v0 + u1 * v1)
    for k in (8, 4, 2, 1):
      nxt = []
      for i in range(len(t) // 2):
        a, b = t[2 * i], t[2 * i + 1]
        ra = a + _take(a, rot[k])
        rb = b + _take(b, rot[k])
        if k == 8:
          nxt.append(jnp.where(keep[k], ra, rb))
        else:
          nxt.append(jnp.where(keep[k], ra, _take(rb, rotr[k])))
      t = nxt
    out_v[pl.ds(r0, L)] = _take(t[0], perm)
    return ()

  lax.fori_loop(0, NGRP, group, (), unroll=False)

  pltpu.sync_copy(out_v, out_hbm.at[pl.ds(base, BPW)])


@jax.jit
def kernel(user, item, user_factors, item_factors):
  mesh = plsc.VectorSubcoreMesh(core_axis_name="c", subcore_axis_name="s")
  run = pl.kernel(
      _sc_body,
      out_type=jax.ShapeDtypeStruct((B,), jnp.float32),
      mesh=mesh,
      scratch_types=[
          pltpu.VMEM((NCHUNK, CHUNK), jnp.int32),
          pltpu.VMEM((NCHUNK, CHUNK), jnp.int32),
          pltpu.VMEM((BPW, F), jnp.float32),
          pltpu.VMEM((BPW, F), jnp.float32),
          pltpu.VMEM((BPW,), jnp.float32),
          pltpu.SemaphoreType.DMA,
      ],
      compiler_params=pltpu.CompilerParams(use_tc_tiling_on_sc=False),
  )
  return run(user, item, user_factors, item_factors)
